# baseline (speedup 1.0000x reference)
import functools

import jax
import jax.numpy as jnp
from jax import lax
from jax.experimental import pallas as pl
from jax.experimental.pallas import tpu as pltpu
from jax.experimental.pallas import tpu_sc as plsc

N = 10000
E = 320000
C = 128

NPAD = 10240
NCORES = 2
NSUB = 16
NW = NCORES * NSUB
EPW = E // NW
CHUNK = 32
NCHUNK = EPW // CHUNK
TB = EPW - NCHUNK * CHUNK
NITER = NCHUNK // 2
RPW = NPAD // NSUB
L = 16


def _prep_body(x_ref, w1_ref, ts_ref, td_ref):
    xb = x_ref[...]
    w1 = w1_ref[...]
    ts_ref[:, :C] = jnp.dot(xb, w1[:C], preferred_element_type=jnp.float32)
    ts_ref[:, C:] = xb
    td_ref[...] = jnp.dot(xb, w1[C:], preferred_element_type=jnp.float32)


def _precompute(x, W1):
    blk = 1000
    grid = (N // blk,)
    return pl.pallas_call(
        _prep_body,
        grid=grid,
        in_specs=[
            pl.BlockSpec((blk, C), lambda i: (i, 0)),
            pl.BlockSpec((2 * C, C), lambda i: (0, 0)),
        ],
        out_specs=[
            pl.BlockSpec((blk, 2 * C), lambda i: (i, 0)),
            pl.BlockSpec((blk, C), lambda i: (i, 0)),
        ],
        out_shape=[
            jax.ShapeDtypeStruct((N, 2 * C), jnp.float32),
            jax.ShapeDtypeStruct((N, C), jnp.float32),
        ],
    )(x, W1)


def _dve_body(d_ref, de_ref, w2_ref, out_ref):
    dv20 = jnp.dot(de_ref[...], w2_ref[...][C:],
                   preferred_element_type=jnp.float32)
    db = d_ref[...].reshape(E // C, C) // 50
    val = jnp.full(db.shape, dv20[19, 0], jnp.float32)
    for b in range(19):
        val = jnp.where(db == b, dv20[b, 0], val)
    out_ref[...] = val.reshape(E)


def _dval_edges(distances, dist_emb, W2):
    return pl.pallas_call(
        _dve_body,
        grid=(1,),
        in_specs=[
            pl.BlockSpec((E,), lambda i: (0,)),
            pl.BlockSpec((20, C), lambda i: (0, 0)),
            pl.BlockSpec((2 * C, 1), lambda i: (0, 0)),
        ],
        out_specs=pl.BlockSpec((E,), lambda i: (0,)),
        out_shape=jax.ShapeDtypeStruct((E,), jnp.float32),
    )(distances, dist_emb, W2)


def _sc_body(ts_hbm, td_hbm, ei_hbm, dve_hbm, w2a_hbm,
             agg_out, cnt_out,
             rows_sA, rows_sB, rows_dA, rows_dB, wbufA, wbufB,
             attbA, attbB, ibufA, ibufB, dvebA, dvebB, dscatA, dscatB,
             ibufT, dvebT, dscatT, w2a_v, agg_sh, cnt_sh,
             gsemA, gsemB, ssemA, ssemB, isemA, isemB):
    cid = lax.axis_index("c")
    sid = lax.axis_index("s")
    wid = cid * NSUB + sid
    ebase = wid * EPW

    zeros16 = jnp.zeros((L,), jnp.float32)

    def zrow(r, carry):
        for j in range(C // L):
            wbufA[r, j * L:(j + 1) * L] = zeros16
        attbA[r, 0:L] = zeros16
        return carry
    lax.fori_loop(0, CHUNK, zrow, 0)
    for k in range(RPW // CHUNK):
        off = sid * RPW + k * CHUNK
        pltpu.sync_copy(wbufA, agg_sh.at[pl.ds(off, CHUNK)])
        pltpu.sync_copy(attbA, cnt_sh.at[pl.ds(off, CHUNK)])

    pltpu.sync_copy(w2a_hbm, w2a_v)
    plsc.subcore_barrier()

    w2a_vecs = [w2a_v[j * L:(j + 1) * L] for j in range(C // L)]
    iota16 = lax.iota(jnp.int32, L)

    def idx_issue(c, ibuf, dveb, isem):
        base = pl.multiple_of(ebase + c * CHUNK, CHUNK)
        pltpu.async_copy(ei_hbm.at[pl.ds(0, 2), pl.ds(base, CHUNK)],
                         ibuf, isem)
        pltpu.async_copy(dve_hbm.at[pl.ds(base, CHUNK)], dveb, isem)

    def idx_wait(ibuf, dveb, isem):
        pltpu.make_async_copy(
            ei_hbm.at[pl.ds(0, 2), pl.ds(0, CHUNK)], ibuf, isem).wait()
        pltpu.make_async_copy(dve_hbm.at[pl.ds(0, CHUNK)], dveb, isem).wait()

    def gather_issue(ibuf, rs, rd, gsem):
        pltpu.async_copy(ts_hbm.at[ibuf.at[0]], rs, gsem)
        pltpu.async_copy(td_hbm.at[ibuf.at[1]], rd, gsem)

    def gather_wait(rs, rd, gsem):
        pltpu.make_async_copy(ts_hbm.at[pl.ds(0, CHUNK)], rs, gsem).wait()
        pltpu.make_async_copy(td_hbm.at[pl.ds(0, CHUNK)], rd, gsem).wait()

    def scatter_issue(wb, ab, dscat, ssem):
        pltpu.async_copy(wb, agg_sh.at[dscat], ssem, add=True)
        pltpu.async_copy(ab, cnt_sh.at[dscat], ssem, add=True)

    def scatter_wait(wb, ab, ssem):
        pltpu.make_async_copy(
            ts_hbm.at[pl.ds(0, CHUNK), pl.ds(0, C)], wb, ssem).wait()
        pltpu.make_async_copy(
            ts_hbm.at[pl.ds(0, CHUNK), pl.ds(0, L)], ab, ssem).wait()

    def do_group(rs, rd, dv, wb, ab, e0):
        s_sc = []
        for ee in range(L):
            e = e0 + ee
            acc = zeros16
            for j in range(C // L):
                ga = rs[e, j * L:(j + 1) * L]
                gb = rd[e, j * L:(j + 1) * L]
                h = ga + gb
                lr = jnp.maximum(h, 0.2 * h)
                acc = acc + lr * w2a_vecs[j]
            s_sc.append(jnp.sum(acc))
        logits = jnp.full((L,), s_sc[0], jnp.float32)
        for ee in range(1, L):
            logits = jnp.where(iota16 == ee, s_sc[ee], logits)
        logits = logits + dv
        sg = 1.0 / (1.0 + jnp.exp(-logits))
        att = jnp.exp(sg)
        for ee in range(L):
            e = e0 + ee
            attbc = jnp.full((L,), att[ee], jnp.float32)
            for j in range(C // L):
                wb[e, j * L:(j + 1) * L] = rs[e, C + j * L:C + (j + 1) * L] * attbc
            ab[e, 0:L] = attbc

    def compute_chunk(ibuf, dveb, rs, rd, wb, ab, dscat):
        for j in range(CHUNK // L):
            dscat[j * L:(j + 1) * L] = ibuf[1, j * L:(j + 1) * L]
        for g in range(CHUNK // L):
            do_group(rs, rd, dveb[g * L:(g + 1) * L], wb, ab, g * L)

    pltpu.sync_copy(ei_hbm.at[pl.ds(0, 2),
                              pl.ds(pl.multiple_of(ebase, CHUNK), CHUNK)],
                    ibufA)
    pltpu.sync_copy(dve_hbm.at[pl.ds(pl.multiple_of(ebase, CHUNK), CHUNK)],
                    dvebA)
    gather_issue(ibufA, rows_sA, rows_dA, gsemA)
    idx_issue(1, ibufB, dvebB, isemB)

    def pipe(k, carry):
        idx_wait(ibufB, dvebB, isemB)
        gather_issue(ibufB, rows_sB, rows_dB, gsemB)
        gather_wait(rows_sA, rows_dA, gsemA)

        @pl.when(k > 0)
        def _():
            scatter_wait(wbufA, attbA, ssemA)
        compute_chunk(ibufA, dvebA, rows_sA, rows_dA, wbufA, attbA, dscatA)
        scatter_issue(wbufA, attbA, dscatA, ssemA)

        @pl.when(k < NITER - 1)
        def _():
            idx_issue(2 * k + 2, ibufA, dvebA, isemA)

        @pl.when(k < NITER - 1)
        def _():
            idx_wait(ibufA, dvebA, isemA)
            gather_issue(ibufA, rows_sA, rows_dA, gsemA)
        gather_wait(rows_sB, rows_dB, gsemB)

        @pl.when(k > 0)
        def _():
            scatter_wait(wbufB, attbB, ssemB)
        compute_chunk(ibufB, dvebB, rows_sB, rows_dB, wbufB, attbB, dscatB)
        scatter_issue(wbufB, attbB, dscatB, ssemB)

        @pl.when(k < NITER - 1)
        def _():
            idx_issue(2 * k + 3, ibufB, dvebB, isemB)
        return carry
    lax.fori_loop(0, NITER, pipe, 0)

    scatter_wait(wbufA, attbA, ssemA)
    scatter_wait(wbufB, attbB, ssemB)

    tbase = pl.multiple_of(ebase + NCHUNK * CHUNK, TB)
    pltpu.sync_copy(ei_hbm.at[pl.ds(0, 2), pl.ds(tbase, TB)], ibufT)
    pltpu.sync_copy(dve_hbm.at[pl.ds(tbase, TB)], dvebT)
    pltpu.async_copy(ts_hbm.at[ibufT.at[0]],
                     rows_sB.at[pl.ds(0, TB)], gsemB)
    pltpu.async_copy(td_hbm.at[ibufT.at[1]],
                     rows_dB.at[pl.ds(0, TB)], gsemB)
    pltpu.make_async_copy(ts_hbm.at[pl.ds(0, TB)],
                          rows_sB.at[pl.ds(0, TB)], gsemB).wait()
    pltpu.make_async_copy(ts_hbm.at[pl.ds(0, TB), pl.ds(0, C)],
                          rows_dB.at[pl.ds(0, TB)], gsemB).wait()
    dscatT[0:L] = ibufT[1, 0:L]
    do_group(rows_sB, rows_dB, dvebT[0:L], wbufB, attbB, 0)
    pltpu.sync_copy(wbufB.at[pl.ds(0, TB)], agg_sh.at[dscatT], add=True)
    pltpu.sync_copy(attbB.at[pl.ds(0, TB)], cnt_sh.at[dscatT], add=True)

    plsc.subcore_barrier()
    out_off = sid * RPW
    pltpu.sync_copy(agg_sh.at[pl.ds(out_off, RPW)],
                    agg_out.at[cid, pl.ds(out_off, RPW)])
    pltpu.sync_copy(cnt_sh.at[pl.ds(out_off, RPW)],
                    cnt_out.at[cid, pl.ds(out_off, RPW)])


def _sc_edges(table_src, table_dst, edge_index, dve, w2a):
    mesh = plsc.VectorSubcoreMesh(core_axis_name="c", subcore_axis_name="s",
                                  num_cores=NCORES)
    f = pl.kernel(
        _sc_body,
        out_type=[
            jax.ShapeDtypeStruct((NCORES, NPAD, C), jnp.float32),
            jax.ShapeDtypeStruct((NCORES, NPAD, L), jnp.float32),
        ],
        mesh=mesh,
        compiler_params=pltpu.CompilerParams(needs_layout_passes=False,
                                             use_tc_tiling_on_sc=False),
        scratch_types=[
            pltpu.VMEM((CHUNK, 2 * C), jnp.float32),
            pltpu.VMEM((CHUNK, 2 * C), jnp.float32),
            pltpu.VMEM((CHUNK, C), jnp.float32),
            pltpu.VMEM((CHUNK, C), jnp.float32),
            pltpu.VMEM((CHUNK, C), jnp.float32),
            pltpu.VMEM((CHUNK, C), jnp.float32),
            pltpu.VMEM((CHUNK, L), jnp.float32),
            pltpu.VMEM((CHUNK, L), jnp.float32),
            pltpu.VMEM((2, CHUNK), jnp.int32),
            pltpu.VMEM((2, CHUNK), jnp.int32),
            pltpu.VMEM((CHUNK,), jnp.float32),
            pltpu.VMEM((CHUNK,), jnp.float32),
            pltpu.VMEM((CHUNK,), jnp.int32),
            pltpu.VMEM((CHUNK,), jnp.int32),
            pltpu.VMEM((2, TB), jnp.int32),
            pltpu.VMEM((TB,), jnp.float32),
            pltpu.VMEM((TB,), jnp.int32),
            pltpu.VMEM((C,), jnp.float32),
            pltpu.VMEM_SHARED((NPAD, C), jnp.float32),
            pltpu.VMEM_SHARED((NPAD, L), jnp.float32),
            pltpu.SemaphoreType.DMA,
            pltpu.SemaphoreType.DMA,
            pltpu.SemaphoreType.DMA,
            pltpu.SemaphoreType.DMA,
            pltpu.SemaphoreType.DMA,
            pltpu.SemaphoreType.DMA,
        ],
    )
    return f(table_src, table_dst, edge_index, dve, w2a)


def _fin_body(agg_ref, cnt_ref, out_ref):
    a = agg_ref[0]
    c = cnt_ref[0, :, 0:1]
    for k in range(1, NCORES):
        a = a + agg_ref[k]
        c = c + cnt_ref[k, :, 0:1]
    out_ref[...] = jnp.maximum(a / (c + 1e-6), 0.0)


def _finalize(agg, cnt):
    blk = 2000
    grid = (N // blk,)
    return pl.pallas_call(
        _fin_body,
        grid=grid,
        in_specs=[
            pl.BlockSpec((NCORES, blk, C), lambda i: (0, i, 0)),
            pl.BlockSpec((NCORES, blk, L), lambda i: (0, i, 0)),
        ],
        out_specs=pl.BlockSpec((blk, C), lambda i: (i, 0)),
        out_shape=jax.ShapeDtypeStruct((N, C), jnp.float32),
    )(agg, cnt)


def kernel(x, edge_index, distances, W1, W2, dist_emb):
    table_src, table_dst = _precompute(x, W1)
    dve = _dval_edges(distances, dist_emb, W2)
    w2a = W2[:C, 0]
    agg, cnt = _sc_edges(table_src, table_dst, edge_index, dve, w2a)
    return _finalize(agg, cnt)


# speedup vs baseline: 1.6201x; 1.6201x over previous
import functools

import jax
import jax.numpy as jnp
from jax import lax
from jax.experimental import pallas as pl
from jax.experimental.pallas import tpu as pltpu
from jax.experimental.pallas import tpu_sc as plsc

N = 10000
E = 320000
C = 128

NPAD = 10240
NCORES = 2
NSUB = 16
NW = NCORES * NSUB
EPW = E // NW
CHUNK = 32
NCHUNK = EPW // CHUNK
TB = EPW - NCHUNK * CHUNK
NITER = NCHUNK // 2
RPW = NPAD // NSUB
L = 16


H = C // 2


def _pack_cols(v):
    lo = lax.bitcast_convert_type(v[..., :H].astype(jnp.bfloat16), jnp.uint16)
    hi = lax.bitcast_convert_type(v[..., H:].astype(jnp.bfloat16), jnp.uint16)
    w = lo.astype(jnp.uint32) | (hi.astype(jnp.uint32) << 16)
    return lax.bitcast_convert_type(w, jnp.float32)


def _prep_body(x_ref, w1_ref, ts_ref, td_ref):
    xb = x_ref[...]
    w1 = w1_ref[...]
    xa = jnp.dot(xb, w1[:C], preferred_element_type=jnp.float32)
    ts_ref[:, :H] = _pack_cols(xa)
    ts_ref[:, H:] = xb
    td_ref[...] = _pack_cols(
        jnp.dot(xb, w1[C:], preferred_element_type=jnp.float32))


def _precompute(x, W1):
    blk = 1000
    grid = (N // blk,)
    return pl.pallas_call(
        _prep_body,
        grid=grid,
        in_specs=[
            pl.BlockSpec((blk, C), lambda i: (i, 0)),
            pl.BlockSpec((2 * C, C), lambda i: (0, 0)),
        ],
        out_specs=[
            pl.BlockSpec((blk, 3 * H), lambda i: (i, 0)),
            pl.BlockSpec((blk, H), lambda i: (i, 0)),
        ],
        out_shape=[
            jax.ShapeDtypeStruct((N, 3 * H), jnp.float32),
            jax.ShapeDtypeStruct((N, H), jnp.float32),
        ],
    )(x, W1)


def _dve_body(d_ref, de_ref, w2_ref, out_ref, w2a_ref):
    w2 = w2_ref[...]
    dv20 = jnp.dot(de_ref[...], w2[C:], preferred_element_type=jnp.float32)
    db = d_ref[...].reshape(E // C, C) // 50
    val = jnp.full(db.shape, dv20[19, 0], jnp.float32)
    for b in range(19):
        val = jnp.where(db == b, dv20[b, 0], val)
    out_ref[...] = val.reshape(E)
    w2a_ref[...] = _pack_cols(w2[:C].reshape(1, C))


def _dval_edges(distances, dist_emb, W2):
    return pl.pallas_call(
        _dve_body,
        grid=(1,),
        in_specs=[
            pl.BlockSpec((E,), lambda i: (0,)),
            pl.BlockSpec((20, C), lambda i: (0, 0)),
            pl.BlockSpec((2 * C, 1), lambda i: (0, 0)),
        ],
        out_specs=[
            pl.BlockSpec((E,), lambda i: (0,)),
            pl.BlockSpec((1, H), lambda i: (0, 0)),
        ],
        out_shape=[
            jax.ShapeDtypeStruct((E,), jnp.float32),
            jax.ShapeDtypeStruct((1, H), jnp.float32),
        ],
    )(distances, dist_emb, W2)


def _sc_body(ts_hbm, td_hbm, ei_hbm, dve_hbm, w2a_hbm,
             agg_out, cnt_out,
             rows_sA, rows_sB, rows_dA, rows_dB, wbufA, wbufB,
             attbA, attbB, ibufA, ibufB, dvebA, dvebB, dscatA, dscatB,
             ibufT, dvebT, dscatT, w2a_v, agg_sh, cnt_sh,
             gsemA, gsemB, ssemA, ssemB, isemA, isemB):
    cid = lax.axis_index("c")
    sid = lax.axis_index("s")
    wid = cid * NSUB + sid
    ebase = wid * EPW

    zeros16 = jnp.zeros((L,), jnp.float32)

    def zrow(r, carry):
        for j in range(C // L):
            wbufA[r, j * L:(j + 1) * L] = zeros16
        attbA[r, 0:L] = zeros16
        return carry
    lax.fori_loop(0, CHUNK, zrow, 0)
    for k in range(RPW // CHUNK):
        off = sid * RPW + k * CHUNK
        pltpu.sync_copy(wbufA, agg_sh.at[pl.ds(off, CHUNK)])
        pltpu.sync_copy(attbA, cnt_sh.at[pl.ds(off, CHUNK)])

    pltpu.sync_copy(w2a_hbm, w2a_v)
    plsc.subcore_barrier()

    w2a_vecs = [plsc.bitcast(w2a_v[0, j * L:(j + 1) * L], jnp.bfloat16)
                for j in range(H // L)]
    iota16 = lax.iota(jnp.int32, L)

    def idx_issue(c, ibuf, dveb, isem):
        base = pl.multiple_of(ebase + c * CHUNK, CHUNK)
        pltpu.async_copy(ei_hbm.at[pl.ds(0, 2), pl.ds(base, CHUNK)],
                         ibuf, isem)
        pltpu.async_copy(dve_hbm.at[pl.ds(base, CHUNK)], dveb, isem)

    def idx_wait(ibuf, dveb, isem):
        pltpu.make_async_copy(
            ei_hbm.at[pl.ds(0, 2), pl.ds(0, CHUNK)], ibuf, isem).wait()
        pltpu.make_async_copy(dve_hbm.at[pl.ds(0, CHUNK)], dveb, isem).wait()

    def gather_issue(ibuf, rs, rd, gsem):
        pltpu.async_copy(ts_hbm.at[ibuf.at[0]], rs, gsem)
        pltpu.async_copy(td_hbm.at[ibuf.at[1]], rd, gsem)

    def gather_wait(rs, rd, gsem):
        pltpu.make_async_copy(ts_hbm.at[pl.ds(0, CHUNK)], rs, gsem).wait()
        pltpu.make_async_copy(td_hbm.at[pl.ds(0, CHUNK)], rd, gsem).wait()

    def scatter_issue(wb, ab, dscat, ssem):
        pltpu.async_copy(wb, agg_sh.at[dscat], ssem, add=True)
        pltpu.async_copy(ab, cnt_sh.at[dscat], ssem, add=True)

    def scatter_wait(wb, ab, ssem):
        pltpu.make_async_copy(
            ts_hbm.at[pl.ds(0, CHUNK), pl.ds(0, C)], wb, ssem).wait()
        pltpu.make_async_copy(
            ts_hbm.at[pl.ds(0, CHUNK), pl.ds(0, L)], ab, ssem).wait()

    def do_group(rs, rd, dv, wb, ab, e0):
        s_sc = []
        for ee in range(L):
            e = e0 + ee
            acc = zeros16
            for j in range(H // L):
                ga = plsc.bitcast(rs[e, j * L:(j + 1) * L], jnp.bfloat16)
                gb = plsc.bitcast(rd[e, j * L:(j + 1) * L], jnp.bfloat16)
                h = ga + gb
                lr = jnp.maximum(h, h * jnp.bfloat16(0.2))
                p = lr * w2a_vecs[j]
                pa, pb = plsc.unpack(p, format=plsc.PackFormat.INTERLEAVED)
                acc = acc + pa + pb
            s_sc.append(jnp.sum(acc))
        logits = jnp.full((L,), s_sc[0], jnp.float32)
        for ee in range(1, L):
            logits = jnp.where(iota16 == ee, s_sc[ee], logits)
        logits = logits + dv
        sg = 1.0 / (1.0 + jnp.exp(-logits))
        att = jnp.exp(sg)
        for ee in range(L):
            e = e0 + ee
            attbc = jnp.full((L,), att[ee], jnp.float32)
            for j in range(C // L):
                wb[e, j * L:(j + 1) * L] = rs[e, H + j * L:H + (j + 1) * L] * attbc
            ab[e, 0:L] = attbc

    def compute_chunk(ibuf, dveb, rs, rd, wb, ab, dscat):
        for j in range(CHUNK // L):
            dscat[j * L:(j + 1) * L] = ibuf[1, j * L:(j + 1) * L]
        for g in range(CHUNK // L):
            do_group(rs, rd, dveb[g * L:(g + 1) * L], wb, ab, g * L)

    pltpu.sync_copy(ei_hbm.at[pl.ds(0, 2),
                              pl.ds(pl.multiple_of(ebase, CHUNK), CHUNK)],
                    ibufA)
    pltpu.sync_copy(dve_hbm.at[pl.ds(pl.multiple_of(ebase, CHUNK), CHUNK)],
                    dvebA)
    gather_issue(ibufA, rows_sA, rows_dA, gsemA)
    idx_issue(1, ibufB, dvebB, isemB)

    def pipe(k, carry):
        idx_wait(ibufB, dvebB, isemB)
        gather_issue(ibufB, rows_sB, rows_dB, gsemB)
        gather_wait(rows_sA, rows_dA, gsemA)

        @pl.when(k > 0)
        def _():
            scatter_wait(wbufA, attbA, ssemA)
        compute_chunk(ibufA, dvebA, rows_sA, rows_dA, wbufA, attbA, dscatA)
        scatter_issue(wbufA, attbA, dscatA, ssemA)

        @pl.when(k < NITER - 1)
        def _():
            idx_issue(2 * k + 2, ibufA, dvebA, isemA)

        @pl.when(k < NITER - 1)
        def _():
            idx_wait(ibufA, dvebA, isemA)
            gather_issue(ibufA, rows_sA, rows_dA, gsemA)
        gather_wait(rows_sB, rows_dB, gsemB)

        @pl.when(k > 0)
        def _():
            scatter_wait(wbufB, attbB, ssemB)
        compute_chunk(ibufB, dvebB, rows_sB, rows_dB, wbufB, attbB, dscatB)
        scatter_issue(wbufB, attbB, dscatB, ssemB)

        @pl.when(k < NITER - 1)
        def _():
            idx_issue(2 * k + 3, ibufB, dvebB, isemB)
        return carry
    lax.fori_loop(0, NITER, pipe, 0)

    scatter_wait(wbufA, attbA, ssemA)
    scatter_wait(wbufB, attbB, ssemB)

    tbase = pl.multiple_of(ebase + NCHUNK * CHUNK, TB)
    pltpu.sync_copy(ei_hbm.at[pl.ds(0, 2), pl.ds(tbase, TB)], ibufT)
    pltpu.sync_copy(dve_hbm.at[pl.ds(tbase, TB)], dvebT)
    pltpu.async_copy(ts_hbm.at[ibufT.at[0]],
                     rows_sB.at[pl.ds(0, TB)], gsemB)
    pltpu.async_copy(td_hbm.at[ibufT.at[1]],
                     rows_dB.at[pl.ds(0, TB)], gsemB)
    pltpu.make_async_copy(ts_hbm.at[pl.ds(0, TB)],
                          rows_sB.at[pl.ds(0, TB)], gsemB).wait()
    pltpu.make_async_copy(td_hbm.at[pl.ds(0, TB)],
                          rows_dB.at[pl.ds(0, TB)], gsemB).wait()
    dscatT[0:L] = ibufT[1, 0:L]
    do_group(rows_sB, rows_dB, dvebT[0:L], wbufB, attbB, 0)
    pltpu.sync_copy(wbufB.at[pl.ds(0, TB)], agg_sh.at[dscatT], add=True)
    pltpu.sync_copy(attbB.at[pl.ds(0, TB)], cnt_sh.at[dscatT], add=True)

    plsc.subcore_barrier()
    out_off = sid * RPW
    pltpu.sync_copy(agg_sh.at[pl.ds(out_off, RPW)],
                    agg_out.at[cid, pl.ds(out_off, RPW)])
    pltpu.sync_copy(cnt_sh.at[pl.ds(out_off, RPW)],
                    cnt_out.at[cid, pl.ds(out_off, RPW)])


def _sc_edges(table_src, table_dst, edge_index, dve, w2a):
    mesh = plsc.VectorSubcoreMesh(core_axis_name="c", subcore_axis_name="s",
                                  num_cores=NCORES)
    f = pl.kernel(
        _sc_body,
        out_type=[
            jax.ShapeDtypeStruct((NCORES, NPAD, C), jnp.float32),
            jax.ShapeDtypeStruct((NCORES, NPAD, L), jnp.float32),
        ],
        mesh=mesh,
        compiler_params=pltpu.CompilerParams(needs_layout_passes=False,
                                             use_tc_tiling_on_sc=False),
        scratch_types=[
            pltpu.VMEM((CHUNK, 3 * H), jnp.float32),
            pltpu.VMEM((CHUNK, 3 * H), jnp.float32),
            pltpu.VMEM((CHUNK, H), jnp.float32),
            pltpu.VMEM((CHUNK, H), jnp.float32),
            pltpu.VMEM((CHUNK, C), jnp.float32),
            pltpu.VMEM((CHUNK, C), jnp.float32),
            pltpu.VMEM((CHUNK, L), jnp.float32),
            pltpu.VMEM((CHUNK, L), jnp.float32),
            pltpu.VMEM((2, CHUNK), jnp.int32),
            pltpu.VMEM((2, CHUNK), jnp.int32),
            pltpu.VMEM((CHUNK,), jnp.float32),
            pltpu.VMEM((CHUNK,), jnp.float32),
            pltpu.VMEM((CHUNK,), jnp.int32),
            pltpu.VMEM((CHUNK,), jnp.int32),
            pltpu.VMEM((2, TB), jnp.int32),
            pltpu.VMEM((TB,), jnp.float32),
            pltpu.VMEM((TB,), jnp.int32),
            pltpu.VMEM((1, H), jnp.float32),
            pltpu.VMEM_SHARED((NPAD, C), jnp.float32),
            pltpu.VMEM_SHARED((NPAD, L), jnp.float32),
            pltpu.SemaphoreType.DMA,
            pltpu.SemaphoreType.DMA,
            pltpu.SemaphoreType.DMA,
            pltpu.SemaphoreType.DMA,
            pltpu.SemaphoreType.DMA,
            pltpu.SemaphoreType.DMA,
        ],
    )
    return f(table_src, table_dst, edge_index, dve, w2a)


def _fin_body(agg_ref, cnt_ref, out_ref):
    a = agg_ref[0]
    c = cnt_ref[0, :, 0:1]
    for k in range(1, NCORES):
        a = a + agg_ref[k]
        c = c + cnt_ref[k, :, 0:1]
    out_ref[...] = jnp.maximum(a / (c + 1e-6), 0.0)


def _finalize(agg, cnt):
    blk = 2000
    grid = (N // blk,)
    return pl.pallas_call(
        _fin_body,
        grid=grid,
        in_specs=[
            pl.BlockSpec((NCORES, blk, C), lambda i: (0, i, 0)),
            pl.BlockSpec((NCORES, blk, L), lambda i: (0, i, 0)),
        ],
        out_specs=pl.BlockSpec((blk, C), lambda i: (i, 0)),
        out_shape=jax.ShapeDtypeStruct((N, C), jnp.float32),
    )(agg, cnt)


def kernel(x, edge_index, distances, W1, W2, dist_emb):
    table_src, table_dst = _precompute(x, W1)
    dve, w2a_pk = _dval_edges(distances, dist_emb, W2)
    agg, cnt = _sc_edges(table_src, table_dst, edge_index, dve, w2a_pk)
    return _finalize(agg, cnt)
